# Initial kernel scaffold; baseline (speedup 1.0000x reference)
import jax
import jax.numpy as jnp
from jax.experimental import pallas as pl


def kernel(gobyGenotypeIndex, isIndel, matchesReference, fromSequence, toSequence, genotypeCountForwardStrand, genotypeCountReverseStrand, geno_table, bool_table, base_table, count_table, W_ih, W_hh, b_ih, b_hh, W_red, b_red):
    raise NotImplementedError("write your pallas kernel here")


# speedup vs baseline: 1.8654x; 1.8654x over previous
import functools

import jax
import jax.numpy as jnp
from jax import lax
from jax.experimental import pallas as pl
from jax.experimental.pallas import tpu as pltpu
from jax.experimental.pallas import tpu_sc as plsc

B = 4096
L = 20
H = 64
DP = 8

NC = 2
NS = 16
NW = NC * NS

SEQ_N = 2 * B * L
CNT_N = 2 * B
GENO_N = B

_SEQ_PW = SEQ_N // NW
_CNT_PW = CNT_N // NW
_GENO_PW = GENO_N // NW


def _sc_gather(count_t, base_t, geno_t, cnt_idx, seq_idx, geno_idx):
    mesh = plsc.VectorSubcoreMesh(core_axis_name="c", subcore_axis_name="s")

    @functools.partial(
        pl.kernel,
        mesh=mesh,
        compiler_params=pltpu.CompilerParams(use_tc_tiling_on_sc=False),
        out_type=[
            jax.ShapeDtypeStruct((SEQ_N, DP), jnp.float32),
            jax.ShapeDtypeStruct((CNT_N, DP), jnp.float32),
            jax.ShapeDtypeStruct((GENO_N, DP), jnp.float32),
        ],
        scratch_types=[
            pltpu.VMEM((_SEQ_PW,), jnp.int32),
            pltpu.VMEM((_SEQ_PW, DP), jnp.float32),
            pltpu.VMEM((_CNT_PW,), jnp.int32),
            pltpu.VMEM((_CNT_PW, DP), jnp.float32),
            pltpu.VMEM((_GENO_PW,), jnp.int32),
            pltpu.VMEM((_GENO_PW, DP), jnp.float32),
            pltpu.SemaphoreType.DMA,
            pltpu.SemaphoreType.DMA,
            pltpu.SemaphoreType.DMA,
        ],
    )
    def k(count_hbm, base_hbm, geno_hbm, cnt_i_hbm, seq_i_hbm, geno_i_hbm,
          seq_out, cnt_out, geno_out,
          seq_iv, seq_rv, cnt_iv, cnt_rv, geno_iv, geno_rv, s1, s2, s3):
        wid = lax.axis_index("s") * NC + lax.axis_index("c")
        sb = wid * _SEQ_PW
        cb = wid * _CNT_PW
        gb = wid * _GENO_PW
        pltpu.sync_copy(seq_i_hbm.at[pl.ds(sb, _SEQ_PW)], seq_iv)
        pltpu.sync_copy(cnt_i_hbm.at[pl.ds(cb, _CNT_PW)], cnt_iv)
        pltpu.sync_copy(geno_i_hbm.at[pl.ds(gb, _GENO_PW)], geno_iv)
        c1 = pltpu.async_copy(base_hbm.at[seq_iv], seq_rv, s1)
        c2 = pltpu.async_copy(count_hbm.at[cnt_iv], cnt_rv, s2)
        c3 = pltpu.async_copy(geno_hbm.at[geno_iv], geno_rv, s3)
        c1.wait()
        c2.wait()
        c3.wait()
        pltpu.sync_copy(seq_rv, seq_out.at[pl.ds(sb, _SEQ_PW)])
        pltpu.sync_copy(cnt_rv, cnt_out.at[pl.ds(cb, _CNT_PW)])
        pltpu.sync_copy(geno_rv, geno_out.at[pl.ds(gb, _GENO_PW)])

    return k(count_t, base_t, geno_t, cnt_idx, seq_idx, geno_idx)


BB = 512


def _tc_body(seq_ref, cnt_ref, geno_ref, ii_ref, mr_ref, bt_ref,
             wih_ref, whh_ref, b2_ref, wg_ref, wbi_ref, wbm_ref,
             whf_ref, wht_ref, wf_ref, wv_ref, br_ref, out_ref):
    f32 = jnp.float32
    cI = jnp.dot(bt_ref[...], wbi_ref[...], preferred_element_type=f32)
    cM = jnp.dot(bt_ref[...], wbm_ref[...], preferred_element_type=f32)
    ii = ii_ref[...]
    mr = mr_ref[...]
    acc = br_ref[...] + jnp.dot(geno_ref[...], wg_ref[...],
                                preferred_element_type=f32)
    acc = acc + cI[0:1, :] + ii * (cI[1:2, :] - cI[0:1, :])
    acc = acc + cM[0:1, :] + mr * (cM[1:2, :] - cM[0:1, :])
    acc = acc + jnp.dot(cnt_ref[0], wf_ref[...], preferred_element_type=f32)
    acc = acc + jnp.dot(cnt_ref[1], wv_ref[...], preferred_element_type=f32)
    wih = wih_ref[...]
    whh = whh_ref[...]
    b2 = b2_ref[...]
    h0 = jnp.zeros((2 * BB, H), f32)
    c0 = jnp.zeros((2 * BB, H), f32)

    def step(t, carry):
        h, c = carry
        x = seq_ref[t].reshape(2 * BB, DP)
        g = (jnp.dot(x, wih, preferred_element_type=f32)
             + jnp.dot(h, whh, preferred_element_type=f32) + b2)
        gi = jax.nn.sigmoid(g[:, 0:H])
        gf = jax.nn.sigmoid(g[:, H:2 * H])
        gg = jnp.tanh(g[:, 2 * H:3 * H])
        go = jax.nn.sigmoid(g[:, 3 * H:4 * H])
        c = gf * c + gi * gg
        h = go * jnp.tanh(c)
        return h, c

    h, _ = lax.fori_loop(0, L, step, (h0, c0))
    acc = acc + jnp.dot(h[0:BB, :], whf_ref[...], preferred_element_type=f32)
    acc = acc + jnp.dot(h[BB:2 * BB, :], wht_ref[...], preferred_element_type=f32)
    out_ref[...] = jnp.maximum(acc, 0.0)


def kernel(gobyGenotypeIndex, isIndel, matchesReference, fromSequence, toSequence,
           genotypeCountForwardStrand, genotypeCountReverseStrand,
           geno_table, bool_table, base_table, count_table,
           W_ih, W_hh, b_ih, b_hh, W_red, b_red):
    i32 = jnp.int32
    f32 = jnp.float32
    seq_idx = jnp.transpose(
        jnp.stack([fromSequence, toSequence], axis=0).astype(i32),
        (2, 0, 1)).reshape(-1)
    cnt_idx = jnp.concatenate(
        [genotypeCountForwardStrand, genotypeCountReverseStrand],
        axis=0).astype(i32)
    geno_idx = gobyGenotypeIndex.astype(i32)

    count8 = jnp.pad(count_table.astype(f32), ((0, 0), (0, DP - 5)))
    base8 = jnp.pad(base_table.astype(f32), ((0, 0), (0, DP - 6)))
    geno8 = jnp.pad(geno_table.astype(f32), ((0, 0), (0, DP - 4)))

    emb_seq, emb_cnt, emb_geno = _sc_gather(
        count8, base8, geno8, cnt_idx, seq_idx, geno_idx)
    emb_seq = emb_seq.reshape(L, 2, B, DP)
    emb_cnt = emb_cnt.reshape(2, B, DP)

    wih = jnp.pad(W_ih.astype(f32), ((0, 0), (0, DP - 6))).T
    whh = W_hh.astype(f32).T
    b2 = (b_ih + b_hh).astype(f32).reshape(1, 4 * H)
    wr = W_red.astype(f32)
    wg = jnp.pad(wr[0:4], ((0, DP - 4), (0, 0)))
    wbi = wr[4:6]
    wbm = wr[6:8]
    whf = wr[8:72]
    wht = wr[72:136]
    wf = jnp.pad(wr[136:141], ((0, DP - 5), (0, 0)))
    wv = jnp.pad(wr[141:146], ((0, DP - 5), (0, 0)))
    br = b_red.astype(f32).reshape(1, H)
    bt = bool_table.astype(f32)
    ii = isIndel.astype(f32).reshape(B, 1)
    mr = matchesReference.astype(f32).reshape(B, 1)

    const = lambda shape: pl.BlockSpec(shape, lambda i: (0,) * len(shape))
    return pl.pallas_call(
        _tc_body,
        grid=(B // BB,),
        in_specs=[
            pl.BlockSpec((L, 2, BB, DP), lambda i: (0, 0, i, 0)),
            pl.BlockSpec((2, BB, DP), lambda i: (0, i, 0)),
            pl.BlockSpec((BB, DP), lambda i: (i, 0)),
            pl.BlockSpec((BB, 1), lambda i: (i, 0)),
            pl.BlockSpec((BB, 1), lambda i: (i, 0)),
            const((2, 2)),
            const((DP, 4 * H)),
            const((H, 4 * H)),
            const((1, 4 * H)),
            const((DP, H)),
            const((2, H)),
            const((2, H)),
            const((H, H)),
            const((H, H)),
            const((DP, H)),
            const((DP, H)),
            const((1, H)),
        ],
        out_specs=pl.BlockSpec((BB, H), lambda i: (i, 0)),
        out_shape=jax.ShapeDtypeStruct((B, H), f32),
    )(emb_seq, emb_cnt, emb_geno, ii, mr, bt,
      wih, whh, b2, wg, wbi, wbm, whf, wht, wf, wv, br)


# speedup vs baseline: 3.6580x; 1.9610x over previous
import functools

import jax
import jax.numpy as jnp
from jax import lax
from jax.experimental import pallas as pl
from jax.experimental.pallas import tpu as pltpu
from jax.experimental.pallas import tpu_sc as plsc

B = 4096
L = 20
H = 64
DP = 8
BV = 96
GV = 104
BB = 512
NB = B // BB

NC = 2
NS = 16
NW = NC * NS

CNT_N = 2 * B
_CNT_PW = CNT_N // NW


def _sc_gather_count(count_t, cnt_idx):
    mesh = plsc.VectorSubcoreMesh(core_axis_name="c", subcore_axis_name="s")

    @functools.partial(
        pl.kernel,
        mesh=mesh,
        compiler_params=pltpu.CompilerParams(use_tc_tiling_on_sc=False),
        out_type=jax.ShapeDtypeStruct((CNT_N, DP), jnp.float32),
        scratch_types=[
            pltpu.VMEM((_CNT_PW,), jnp.int32),
            pltpu.VMEM((_CNT_PW, DP), jnp.float32),
            pltpu.SemaphoreType.DMA,
        ],
    )
    def k(count_hbm, cnt_i_hbm, cnt_out, cnt_iv, cnt_rv, sem):
        wid = lax.axis_index("s") * NC + lax.axis_index("c")
        cb = wid * _CNT_PW
        pltpu.sync_copy(cnt_i_hbm.at[pl.ds(cb, _CNT_PW)], cnt_iv)
        pltpu.async_copy(count_hbm.at[cnt_iv], cnt_rv, sem).wait()
        pltpu.sync_copy(cnt_rv, cnt_out.at[pl.ds(cb, _CNT_PW)])

    return k(count_t, cnt_idx)


def _tc_body_a(seq_ref, gi_ref, ii_ref, mr_ref, btT_ref,
               wihT_ref, baseT_ref, whh_ref, b2_ref,
               wgT_ref, genoT_ref, wbiT_ref, wbmT_ref,
               whfT_ref, whtT_ref, brc_ref, acc_ref):
    f32 = jnp.float32
    wxT = jnp.dot(wihT_ref[...], baseT_ref[...], preferred_element_type=f32)
    whh = whh_ref[...]
    b2 = b2_ref[...]
    iotaB = lax.broadcasted_iota(jnp.int32, (BV, 2 * BB), 0).astype(f32)

    h0 = jnp.zeros((H, 2 * BB), f32)
    c0 = jnp.zeros((H, 2 * BB), f32)

    def step(t, carry):
        h, c = carry
        oh = (iotaB == seq_ref[t][None, :]).astype(f32)
        g = (jnp.dot(wxT, oh, preferred_element_type=f32)
             + jnp.dot(whh, h, preferred_element_type=f32) + b2)
        gi = jax.nn.sigmoid(g[0:H, :])
        gf = jax.nn.sigmoid(g[H:2 * H, :])
        gg = jnp.tanh(g[2 * H:3 * H, :])
        go = jax.nn.sigmoid(g[3 * H:4 * H, :])
        c = gf * c + gi * gg
        h = go * jnp.tanh(c)
        return h, c

    h, _ = lax.fori_loop(0, L, step, (h0, c0))

    acc = brc_ref[...] + jnp.dot(whfT_ref[...], h[:, 0:BB],
                                 preferred_element_type=f32)
    acc = acc + jnp.dot(whtT_ref[...], h[:, BB:2 * BB],
                        preferred_element_type=f32)
    wgeff = jnp.dot(wgT_ref[...], genoT_ref[...], preferred_element_type=f32)
    iotaG = lax.broadcasted_iota(jnp.int32, (GV, BB), 0).astype(f32)
    ohg = (iotaG == gi_ref[...]).astype(f32)
    acc = acc + jnp.dot(wgeff, ohg, preferred_element_type=f32)
    cIT = jnp.dot(wbiT_ref[...], btT_ref[...], preferred_element_type=f32)
    cMT = jnp.dot(wbmT_ref[...], btT_ref[...], preferred_element_type=f32)
    acc = acc + cIT[:, 0:1] + ii_ref[...] * (cIT[:, 1:2] - cIT[:, 0:1])
    acc = acc + cMT[:, 0:1] + mr_ref[...] * (cMT[:, 1:2] - cMT[:, 0:1])
    acc_ref[...] = acc


def _tc_body_b(acc_ref, cntT_ref, wfT_ref, wvT_ref, out_ref):
    f32 = jnp.float32
    o = (acc_ref[...]
         + jnp.dot(wfT_ref[...], cntT_ref[0], preferred_element_type=f32)
         + jnp.dot(wvT_ref[...], cntT_ref[1], preferred_element_type=f32))
    out_ref[...] = jnp.maximum(o, 0.0)


def kernel(gobyGenotypeIndex, isIndel, matchesReference, fromSequence, toSequence,
           genotypeCountForwardStrand, genotypeCountReverseStrand,
           geno_table, bool_table, base_table, count_table,
           W_ih, W_hh, b_ih, b_hh, W_red, b_red):
    i32 = jnp.int32
    f32 = jnp.float32

    cnt_idx = jnp.concatenate(
        [genotypeCountForwardStrand, genotypeCountReverseStrand],
        axis=0).astype(i32)
    count8 = jnp.pad(count_table.astype(f32), ((0, 0), (0, DP - 5)))
    emb_cnt = _sc_gather_count(count8, cnt_idx)
    cntT = jnp.transpose(emb_cnt.reshape(2, B, DP), (0, 2, 1))

    fr = fromSequence.astype(f32).reshape(NB, BB, L)
    to = toSequence.astype(f32).reshape(NB, BB, L)
    seqf = jnp.stack([fr, to], axis=1).transpose(3, 0, 1, 2).reshape(L, 2 * B)

    gidx = gobyGenotypeIndex.astype(f32).reshape(1, B)
    ii = isIndel.astype(f32).reshape(1, B)
    mr = matchesReference.astype(f32).reshape(1, B)

    wihT = jnp.pad(W_ih.astype(f32), ((0, 0), (0, DP - 6)))
    baseT = jnp.pad(base_table.astype(f32).T, ((0, DP - 6), (0, BV - 85)))
    whh = W_hh.astype(f32)
    b2 = (b_ih + b_hh).astype(f32).reshape(4 * H, 1)
    wr = W_red.astype(f32)
    wgT = jnp.pad(wr[0:4].T, ((0, 0), (0, DP - 4)))
    genoT = jnp.pad(geno_table.astype(f32).T,
                    ((0, DP - 4), (0, GV - 100)))
    wbiT = wr[4:6].T
    wbmT = wr[6:8].T
    whfT = wr[8:72].T
    whtT = wr[72:136].T
    wfT = jnp.pad(wr[136:141].T, ((0, 0), (0, DP - 5)))
    wvT = jnp.pad(wr[141:146].T, ((0, 0), (0, DP - 5)))
    brc = b_red.astype(f32).reshape(H, 1)
    btT = bool_table.astype(f32).T

    const = lambda shape: pl.BlockSpec(shape, lambda i: (0,) * len(shape))
    accT = pl.pallas_call(
        _tc_body_a,
        grid=(NB,),
        in_specs=[
            pl.BlockSpec((L, 2 * BB), lambda i: (0, i)),
            pl.BlockSpec((1, BB), lambda i: (0, i)),
            pl.BlockSpec((1, BB), lambda i: (0, i)),
            pl.BlockSpec((1, BB), lambda i: (0, i)),
            const((2, 2)),
            const((4 * H, DP)),
            const((DP, BV)),
            const((4 * H, H)),
            const((4 * H, 1)),
            const((H, DP)),
            const((DP, GV)),
            const((H, 2)),
            const((H, 2)),
            const((H, H)),
            const((H, H)),
            const((H, 1)),
        ],
        out_specs=pl.BlockSpec((H, BB), lambda i: (0, i)),
        out_shape=jax.ShapeDtypeStruct((H, B), f32),
    )(seqf, gidx, ii, mr, btT, wihT, baseT, whh, b2,
      wgT, genoT, wbiT, wbmT, whfT, whtT, brc)

    outT = pl.pallas_call(
        _tc_body_b,
        out_shape=jax.ShapeDtypeStruct((H, B), f32),
    )(accT, cntT, wfT, wvT)
    return outT.T


# speedup vs baseline: 3.9040x; 1.0672x over previous
import functools

import jax
import jax.numpy as jnp
from jax import lax
from jax.experimental import pallas as pl
from jax.experimental.pallas import tpu as pltpu
from jax.experimental.pallas import tpu_sc as plsc

B = 4096
L = 20
H = 64
DP = 8
BV = 96
GV = 104
BB = 1024
NB = B // BB

NC = 2
NS = 16
NW = NC * NS

CNT_N = 2 * B
_CNT_PW = CNT_N // NW


def _sc_gather_count(count_t, cnt_idx):
    mesh = plsc.VectorSubcoreMesh(core_axis_name="c", subcore_axis_name="s")

    @functools.partial(
        pl.kernel,
        mesh=mesh,
        compiler_params=pltpu.CompilerParams(use_tc_tiling_on_sc=False),
        out_type=jax.ShapeDtypeStruct((CNT_N, DP), jnp.float32),
        scratch_types=[
            pltpu.VMEM((_CNT_PW,), jnp.int32),
            pltpu.VMEM((_CNT_PW, DP), jnp.float32),
            pltpu.SemaphoreType.DMA,
        ],
    )
    def k(count_hbm, cnt_i_hbm, cnt_out, cnt_iv, cnt_rv, sem):
        wid = lax.axis_index("s") * NC + lax.axis_index("c")
        cb = wid * _CNT_PW
        pltpu.sync_copy(cnt_i_hbm.at[pl.ds(cb, _CNT_PW)], cnt_iv)
        pltpu.async_copy(count_hbm.at[cnt_iv], cnt_rv, sem).wait()
        pltpu.sync_copy(cnt_rv, cnt_out.at[pl.ds(cb, _CNT_PW)])

    return k(count_t, cnt_idx)


def _tc_body_a(seq_ref, gi_ref, ii_ref, mr_ref, btT_ref,
               wihT_ref, baseT_ref, whh_ref, b2_ref,
               wgT_ref, genoT_ref, wbiT_ref, wbmT_ref,
               whfT_ref, whtT_ref, brc_ref, acc_ref):
    f32 = jnp.float32
    wxT = jnp.dot(wihT_ref[...], baseT_ref[...], preferred_element_type=f32)
    whh = whh_ref[...]
    b2 = b2_ref[...]
    iotaB = lax.broadcasted_iota(jnp.int32, (BV, 2 * BB), 0).astype(f32)

    h0 = jnp.zeros((H, 2 * BB), f32)
    c0 = jnp.zeros((H, 2 * BB), f32)

    def step(t, carry):
        h, c = carry
        oh = (iotaB == seq_ref[t][None, :]).astype(f32)
        g = (jnp.dot(wxT, oh, preferred_element_type=f32)
             + jnp.dot(whh, h, preferred_element_type=f32) + b2)
        gi = jax.nn.sigmoid(g[0:H, :])
        gf = jax.nn.sigmoid(g[H:2 * H, :])
        gg = jnp.tanh(g[2 * H:3 * H, :])
        go = jax.nn.sigmoid(g[3 * H:4 * H, :])
        c = gf * c + gi * gg
        h = go * jnp.tanh(c)
        return h, c

    h, _ = lax.fori_loop(0, L, step, (h0, c0))

    acc = brc_ref[...] + jnp.dot(whfT_ref[...], h[:, 0:BB],
                                 preferred_element_type=f32)
    acc = acc + jnp.dot(whtT_ref[...], h[:, BB:2 * BB],
                        preferred_element_type=f32)
    wgeff = jnp.dot(wgT_ref[...], genoT_ref[...], preferred_element_type=f32)
    iotaG = lax.broadcasted_iota(jnp.int32, (GV, BB), 0).astype(f32)
    ohg = (iotaG == gi_ref[...]).astype(f32)
    acc = acc + jnp.dot(wgeff, ohg, preferred_element_type=f32)
    cIT = jnp.dot(wbiT_ref[...], btT_ref[...], preferred_element_type=f32)
    cMT = jnp.dot(wbmT_ref[...], btT_ref[...], preferred_element_type=f32)
    acc = acc + cIT[:, 0:1] + ii_ref[...] * (cIT[:, 1:2] - cIT[:, 0:1])
    acc = acc + cMT[:, 0:1] + mr_ref[...] * (cMT[:, 1:2] - cMT[:, 0:1])
    acc_ref[...] = acc


def _tc_body_b(acc_ref, cntT_ref, wfT_ref, wvT_ref, out_ref):
    f32 = jnp.float32
    o = (acc_ref[...]
         + jnp.dot(wfT_ref[...], cntT_ref[0], preferred_element_type=f32)
         + jnp.dot(wvT_ref[...], cntT_ref[1], preferred_element_type=f32))
    out_ref[...] = jnp.maximum(o, 0.0)


def kernel(gobyGenotypeIndex, isIndel, matchesReference, fromSequence, toSequence,
           genotypeCountForwardStrand, genotypeCountReverseStrand,
           geno_table, bool_table, base_table, count_table,
           W_ih, W_hh, b_ih, b_hh, W_red, b_red):
    i32 = jnp.int32
    f32 = jnp.float32

    cnt_idx = jnp.concatenate(
        [genotypeCountForwardStrand, genotypeCountReverseStrand],
        axis=0).astype(i32)
    count8 = jnp.pad(count_table.astype(f32), ((0, 0), (0, DP - 5)))
    emb_cnt = _sc_gather_count(count8, cnt_idx)
    cntT = jnp.transpose(emb_cnt.reshape(2, B, DP), (0, 2, 1))

    fr = fromSequence.astype(f32).reshape(NB, BB, L)
    to = toSequence.astype(f32).reshape(NB, BB, L)
    seqf = jnp.stack([fr, to], axis=1).transpose(3, 0, 1, 2).reshape(L, 2 * B)

    gidx = gobyGenotypeIndex.astype(f32).reshape(1, B)
    ii = isIndel.astype(f32).reshape(1, B)
    mr = matchesReference.astype(f32).reshape(1, B)

    wihT = jnp.pad(W_ih.astype(f32), ((0, 0), (0, DP - 6)))
    baseT = jnp.pad(base_table.astype(f32).T, ((0, DP - 6), (0, BV - 85)))
    whh = W_hh.astype(f32)
    b2 = (b_ih + b_hh).astype(f32).reshape(4 * H, 1)
    wr = W_red.astype(f32)
    wgT = jnp.pad(wr[0:4].T, ((0, 0), (0, DP - 4)))
    genoT = jnp.pad(geno_table.astype(f32).T,
                    ((0, DP - 4), (0, GV - 100)))
    wbiT = wr[4:6].T
    wbmT = wr[6:8].T
    whfT = wr[8:72].T
    whtT = wr[72:136].T
    wfT = jnp.pad(wr[136:141].T, ((0, 0), (0, DP - 5)))
    wvT = jnp.pad(wr[141:146].T, ((0, 0), (0, DP - 5)))
    brc = b_red.astype(f32).reshape(H, 1)
    btT = bool_table.astype(f32).T

    const = lambda shape: pl.BlockSpec(shape, lambda i: (0,) * len(shape))
    accT = pl.pallas_call(
        _tc_body_a,
        grid=(NB,),
        in_specs=[
            pl.BlockSpec((L, 2 * BB), lambda i: (0, i)),
            pl.BlockSpec((1, BB), lambda i: (0, i)),
            pl.BlockSpec((1, BB), lambda i: (0, i)),
            pl.BlockSpec((1, BB), lambda i: (0, i)),
            const((2, 2)),
            const((4 * H, DP)),
            const((DP, BV)),
            const((4 * H, H)),
            const((4 * H, 1)),
            const((H, DP)),
            const((DP, GV)),
            const((H, 2)),
            const((H, 2)),
            const((H, H)),
            const((H, H)),
            const((H, 1)),
        ],
        out_specs=pl.BlockSpec((H, BB), lambda i: (0, i)),
        out_shape=jax.ShapeDtypeStruct((H, B), f32),
    )(seqf, gidx, ii, mr, btT, wihT, baseT, whh, b2,
      wgT, genoT, wbiT, wbmT, whfT, whtT, brc)

    outT = pl.pallas_call(
        _tc_body_b,
        out_shape=jax.ShapeDtypeStruct((H, B), f32),
    )(accT, cntT, wfT, wvT)
    return outT.T


# speedup vs baseline: 3.9451x; 1.0105x over previous
import functools

import jax
import jax.numpy as jnp
from jax import lax
from jax.experimental import pallas as pl
from jax.experimental.pallas import tpu as pltpu
from jax.experimental.pallas import tpu_sc as plsc

B = 4096
L = 20
H = 64
DP = 8
BV = 96
GV = 104
BB = 2048
NB = B // BB

NC = 2
NS = 16
NW = NC * NS

CNT_N = 2 * B
_CNT_PW = CNT_N // NW


def _sc_gather_count(count_t, cnt_idx):
    mesh = plsc.VectorSubcoreMesh(core_axis_name="c", subcore_axis_name="s")

    @functools.partial(
        pl.kernel,
        mesh=mesh,
        compiler_params=pltpu.CompilerParams(use_tc_tiling_on_sc=False),
        out_type=jax.ShapeDtypeStruct((CNT_N, DP), jnp.float32),
        scratch_types=[
            pltpu.VMEM((_CNT_PW,), jnp.int32),
            pltpu.VMEM((_CNT_PW, DP), jnp.float32),
            pltpu.SemaphoreType.DMA,
        ],
    )
    def k(count_hbm, cnt_i_hbm, cnt_out, cnt_iv, cnt_rv, sem):
        wid = lax.axis_index("s") * NC + lax.axis_index("c")
        cb = wid * _CNT_PW
        pltpu.sync_copy(cnt_i_hbm.at[pl.ds(cb, _CNT_PW)], cnt_iv)
        pltpu.async_copy(count_hbm.at[cnt_iv], cnt_rv, sem).wait()
        pltpu.sync_copy(cnt_rv, cnt_out.at[pl.ds(cb, _CNT_PW)])

    return k(count_t, cnt_idx)


def _tc_body_a(seq_ref, gi_ref, ii_ref, mr_ref, btT_ref,
               wihT_ref, baseT_ref, whh_ref, b2_ref,
               wgT_ref, genoT_ref, wbiT_ref, wbmT_ref,
               whfT_ref, whtT_ref, brc_ref, acc_ref):
    f32 = jnp.float32
    wxT = jnp.dot(wihT_ref[...], baseT_ref[...], preferred_element_type=f32)
    whh = whh_ref[...]
    b2 = b2_ref[...]
    iotaB = lax.broadcasted_iota(jnp.int32, (BV, 2 * BB), 0).astype(f32)

    h0 = jnp.zeros((H, 2 * BB), f32)
    c0 = jnp.zeros((H, 2 * BB), f32)

    def step(t, carry):
        h, c = carry
        oh = (iotaB == seq_ref[t][None, :]).astype(f32)
        g = (jnp.dot(wxT, oh, preferred_element_type=f32)
             + jnp.dot(whh, h, preferred_element_type=f32) + b2)
        gi = jax.nn.sigmoid(g[0:H, :])
        gf = jax.nn.sigmoid(g[H:2 * H, :])
        gg = jnp.tanh(g[2 * H:3 * H, :])
        go = jax.nn.sigmoid(g[3 * H:4 * H, :])
        c = gf * c + gi * gg
        h = go * jnp.tanh(c)
        return h, c

    h, _ = lax.fori_loop(0, L, step, (h0, c0))

    acc = brc_ref[...] + jnp.dot(whfT_ref[...], h[:, 0:BB],
                                 preferred_element_type=f32)
    acc = acc + jnp.dot(whtT_ref[...], h[:, BB:2 * BB],
                        preferred_element_type=f32)
    wgeff = jnp.dot(wgT_ref[...], genoT_ref[...], preferred_element_type=f32)
    iotaG = lax.broadcasted_iota(jnp.int32, (GV, BB), 0).astype(f32)
    ohg = (iotaG == gi_ref[...]).astype(f32)
    acc = acc + jnp.dot(wgeff, ohg, preferred_element_type=f32)
    cIT = jnp.dot(wbiT_ref[...], btT_ref[...], preferred_element_type=f32)
    cMT = jnp.dot(wbmT_ref[...], btT_ref[...], preferred_element_type=f32)
    acc = acc + cIT[:, 0:1] + ii_ref[...] * (cIT[:, 1:2] - cIT[:, 0:1])
    acc = acc + cMT[:, 0:1] + mr_ref[...] * (cMT[:, 1:2] - cMT[:, 0:1])
    acc_ref[...] = acc


def _tc_body_b(acc_ref, cntT_ref, wfT_ref, wvT_ref, out_ref):
    f32 = jnp.float32
    o = (acc_ref[...]
         + jnp.dot(wfT_ref[...], cntT_ref[0], preferred_element_type=f32)
         + jnp.dot(wvT_ref[...], cntT_ref[1], preferred_element_type=f32))
    out_ref[...] = jnp.maximum(o, 0.0)


def kernel(gobyGenotypeIndex, isIndel, matchesReference, fromSequence, toSequence,
           genotypeCountForwardStrand, genotypeCountReverseStrand,
           geno_table, bool_table, base_table, count_table,
           W_ih, W_hh, b_ih, b_hh, W_red, b_red):
    i32 = jnp.int32
    f32 = jnp.float32

    cnt_idx = jnp.concatenate(
        [genotypeCountForwardStrand, genotypeCountReverseStrand],
        axis=0).astype(i32)
    count8 = jnp.pad(count_table.astype(f32), ((0, 0), (0, DP - 5)))
    emb_cnt = _sc_gather_count(count8, cnt_idx)
    cntT = jnp.transpose(emb_cnt.reshape(2, B, DP), (0, 2, 1))

    fr = fromSequence.astype(f32).reshape(NB, BB, L)
    to = toSequence.astype(f32).reshape(NB, BB, L)
    seqf = jnp.stack([fr, to], axis=1).transpose(3, 0, 1, 2).reshape(L, 2 * B)

    gidx = gobyGenotypeIndex.astype(f32).reshape(1, B)
    ii = isIndel.astype(f32).reshape(1, B)
    mr = matchesReference.astype(f32).reshape(1, B)

    wihT = jnp.pad(W_ih.astype(f32), ((0, 0), (0, DP - 6)))
    baseT = jnp.pad(base_table.astype(f32).T, ((0, DP - 6), (0, BV - 85)))
    whh = W_hh.astype(f32)
    b2 = (b_ih + b_hh).astype(f32).reshape(4 * H, 1)
    wr = W_red.astype(f32)
    wgT = jnp.pad(wr[0:4].T, ((0, 0), (0, DP - 4)))
    genoT = jnp.pad(geno_table.astype(f32).T,
                    ((0, DP - 4), (0, GV - 100)))
    wbiT = wr[4:6].T
    wbmT = wr[6:8].T
    whfT = wr[8:72].T
    whtT = wr[72:136].T
    wfT = jnp.pad(wr[136:141].T, ((0, 0), (0, DP - 5)))
    wvT = jnp.pad(wr[141:146].T, ((0, 0), (0, DP - 5)))
    brc = b_red.astype(f32).reshape(H, 1)
    btT = bool_table.astype(f32).T

    const = lambda shape: pl.BlockSpec(shape, lambda i: (0,) * len(shape))
    accT = pl.pallas_call(
        _tc_body_a,
        grid=(NB,),
        in_specs=[
            pl.BlockSpec((L, 2 * BB), lambda i: (0, i)),
            pl.BlockSpec((1, BB), lambda i: (0, i)),
            pl.BlockSpec((1, BB), lambda i: (0, i)),
            pl.BlockSpec((1, BB), lambda i: (0, i)),
            const((2, 2)),
            const((4 * H, DP)),
            const((DP, BV)),
            const((4 * H, H)),
            const((4 * H, 1)),
            const((H, DP)),
            const((DP, GV)),
            const((H, 2)),
            const((H, 2)),
            const((H, H)),
            const((H, H)),
            const((H, 1)),
        ],
        out_specs=pl.BlockSpec((H, BB), lambda i: (0, i)),
        out_shape=jax.ShapeDtypeStruct((H, B), f32),
    )(seqf, gidx, ii, mr, btT, wihT, baseT, whh, b2,
      wgT, genoT, wbiT, wbmT, whfT, whtT, brc)

    outT = pl.pallas_call(
        _tc_body_b,
        out_shape=jax.ShapeDtypeStruct((H, B), f32),
    )(accT, cntT, wfT, wvT)
    return outT.T
